# baseline (speedup 1.0000x reference)
import functools

import jax
import jax.numpy as jnp
from jax import lax
from jax.experimental import pallas as pl
from jax.experimental.pallas import tpu as pltpu
from jax.experimental.pallas import tpu_sc as plsc

K_POOL = 1024
_NCHUNK = 512
_C = 128


def _rank_body(keysc_ref, maskc_ref, keysr_ref, maskr_ref, idx_ref, vc_ref):
    b = pl.program_id(0)
    N = keysr_ref.shape[2]
    K = idx_ref.shape[2]
    neg_inf = jnp.float32(-jnp.inf)
    skr = jnp.where(maskr_ref[0] > 0, keysr_ref[0], neg_inf)
    iota_m = lax.broadcasted_iota(jnp.int32, (1, N), 1)
    kiota = lax.broadcasted_iota(jnp.int32, (1, K), 1)
    idx_acc = jnp.zeros((1, K), jnp.int32)
    for i in range(N // _NCHUNK):
        sl = pl.ds(i * _NCHUNK, _NCHUNK)
        skc = jnp.where(maskc_ref[0, sl, :] > 0, keysc_ref[0, sl, :], neg_inf)
        nidx = i * _NCHUNK + lax.broadcasted_iota(jnp.int32, (_NCHUNK, 1), 0)
        before = (skr > skc) | ((skr == skc) & (iota_m < nidx))
        cnt = jnp.sum(before.astype(jnp.int32), axis=1, keepdims=True)
        onehot = cnt == kiota
        gidx = b * N + nidx
        idx_acc = idx_acc + jnp.sum(
            jnp.where(onehot, gidx, 0), axis=0, keepdims=True
        )
    idx_ref[...] = idx_acc.reshape(1, 1, K)
    vc = jnp.sum(maskr_ref[0])
    vc_ref[...] = jnp.full((1, 1, K), vc, jnp.int32)


def _tc_rank(keys, maski):
    B, N = keys.shape
    idx3, vc3 = pl.pallas_call(
        _rank_body,
        grid=(B,),
        in_specs=[
            pl.BlockSpec((1, N, 1), lambda b: (b, 0, 0)),
            pl.BlockSpec((1, N, 1), lambda b: (b, 0, 0)),
            pl.BlockSpec((1, 1, N), lambda b: (b, 0, 0)),
            pl.BlockSpec((1, 1, N), lambda b: (b, 0, 0)),
        ],
        out_specs=[
            pl.BlockSpec((1, 1, K_POOL), lambda b: (b, 0, 0)),
            pl.BlockSpec((1, 1, K_POOL), lambda b: (b, 0, 0)),
        ],
        out_shape=[
            jax.ShapeDtypeStruct((B, 1, K_POOL), jnp.int32),
            jax.ShapeDtypeStruct((B, 1, K_POOL), jnp.int32),
        ],
    )(
        keys.reshape(B, N, 1),
        maski.reshape(B, N, 1),
        keys.reshape(B, 1, N),
        maski.reshape(B, 1, N),
    )
    return idx3.reshape(B, K_POOL), vc3[:, 0, 0]


def _gather_body(nc, rpw, F, table_h, idx_h, vc_h, out_h, idx_v, rows_v, vc_v, sem):
    wid = lax.axis_index("s") * nc + lax.axis_index("c")
    wpb = K_POOL // rpw
    b = wid // wpb
    half = wid - b * wpb
    base = wid * rpw
    pltpu.sync_copy(idx_h.at[wid], idx_v)
    pltpu.sync_copy(vc_h, vc_v)
    vc_b = vc_v[pl.ds(b, 16)][0]
    for c in range(rpw // _C):
        pltpu.async_copy(table_h.at[idx_v.at[c]], rows_v, sem).wait()
        kstart = half * rpw + c * _C
        @pl.when(kstart + _C > vc_b)
        def _zero_tail():
            def zrow(r, carry):
                @pl.when(kstart + r >= vc_b)
                def _z():
                    for j in range(F // 16):
                        rows_v[r, pl.ds(j * 16, 16)] = jnp.zeros((16,), jnp.float32)
                return carry
            lax.fori_loop(0, _C, zrow, 0)
        pltpu.sync_copy(rows_v, out_h.at[pl.ds(base + c * _C, _C)])


def _sc_gather(table, idx_r, vc):
    BN, F = table.shape
    NW, nch, _ = idx_r.shape
    rpw = nch * _C
    mesh = plsc.VectorSubcoreMesh(core_axis_name="c", subcore_axis_name="s")
    body = functools.partial(_gather_body, mesh.num_cores, rpw, F)
    fn = pl.kernel(
        body,
        out_type=jax.ShapeDtypeStruct((NW * rpw, F), jnp.float32),
        mesh=mesh,
        scratch_types=[
            pltpu.VMEM((nch, _C), jnp.int32),
            pltpu.VMEM((_C, F), jnp.float32),
            pltpu.VMEM((2 * 16,), jnp.int32),
            pltpu.SemaphoreType.DMA,
        ],
    )
    return fn(table, idx_r, vc)


def kernel(embeddings, mask):
    B, N, F = embeddings.shape
    keys = embeddings[..., F - 1]
    maski = mask.astype(jnp.int32)
    idx, vc = _tc_rank(keys, maski)
    rows_total = B * K_POOL
    NW = 32
    rpw = rows_total // NW
    idx_r = idx.reshape(NW, rpw // _C, _C)
    vc_pad = jnp.pad(vc, (0, 2 * B - vc.shape[0]))
    out_flat = _sc_gather(embeddings.reshape(B * N, F), idx_r, vc_pad)
    return out_flat.reshape(B, K_POOL, F)
